# baseline (speedup 1.0000x reference)
import functools

import jax
import jax.numpy as jnp
from jax import lax
from jax.experimental import pallas as pl
from jax.experimental.pallas import tpu as pltpu
from jax.experimental.pallas import tpu_sc as plsc

N = 10000
E = 320000
D = 128
G = 64
EPS = 1e-5

NC = 2
NS = 16
NPAD = 10240
K = 128
KB = 2
NCH = 160
EPT = NCH * K
EPAD = NS * EPT
RPT = NPAD // NS
NHALF = N // 2
DUMMYL = NHALF
TROWS = 5016
CAP = EPT + 1024
NCAP = CAP // K
WBC = 104
NB = 400
NBLK = N // NB
NB2 = 200
NBLK2 = NHALF // NB2


def _deg_body(dst_hbm, ones_hbm, zeros_hbm, out_hbm,
              dst_v, ones_v, buf_v, deg_sh):
    c = lax.axis_index("c")
    s = lax.axis_index("s")
    pltpu.sync_copy(dst_hbm.at[s], dst_v)
    pltpu.sync_copy(ones_hbm, ones_v)
    pltpu.sync_copy(zeros_hbm, buf_v)

    pltpu.sync_copy(buf_v, deg_sh.at[pl.ds(s * RPT, RPT)])
    plsc.subcore_barrier()

    half = NCH // NC

    def chunk(j, carry):
        pltpu.sync_copy(ones_v, deg_sh.at[dst_v.at[j]], add=True)
        return carry
    lax.fori_loop(c * half, (c + 1) * half, chunk, 0)
    plsc.subcore_barrier()

    pltpu.sync_copy(deg_sh.at[pl.ds(s * RPT, RPT)], buf_v)
    pltpu.sync_copy(buf_v, out_hbm.at[pl.ds(c * NPAD + s * RPT, RPT)])


def _route_body(src_hbm, dst_hbm, souts, douts, counts,
                src_t, dst_t, l0s, l0d, l1s, l1d, cnt_v):
    c = lax.axis_index("c")
    s = lax.axis_index("s")

    @pl.when(c == 0)
    def _():
        pltpu.sync_copy(src_hbm.at[s], src_t)
        pltpu.sync_copy(dst_hbm.at[s], dst_t)

        def chunk(j, carry):
            o0, o1 = carry
            sv = src_t[pl.ds(j * 16, 16)]
            dv = dst_t[pl.ds(j * 16, 16)]
            key = (dv >= NHALF).astype(jnp.int32)
            val = sv * 16384 + dv
            ks, vs = plsc.sort_key_val(key, val)
            l0s[pl.ds(o0, 16)] = vs // 16384
            l0d[pl.ds(o0, 16)] = vs % 16384
            rv = lax.rev(vs, (0,))
            l1s[pl.ds(o1, 16)] = rv // 16384
            l1d[pl.ds(o1, 16)] = rv % 16384 - NHALF
            n1 = jnp.sum(ks)
            return (o0 + (16 - n1), o1 + n1)

        o0, o1 = lax.fori_loop(
            0, EPT // 16, chunk, (jnp.int32(0), jnp.int32(0)))

        zero16 = jnp.zeros((16,), jnp.int32)
        dummy16 = jnp.full((16,), DUMMYL, jnp.int32) + s

        def padloop(q, carry):
            a, b = carry
            l0s[pl.ds(a, 16)] = zero16
            l0d[pl.ds(a, 16)] = dummy16
            l1s[pl.ds(b, 16)] = zero16
            l1d[pl.ds(b, 16)] = dummy16
            return (a + 16, b + 16)
        lax.fori_loop(0, 1024 // 16, padloop, (o0, o1))

        cp0 = ((o0 + 1023) // 1024) * 1024
        cp1 = ((o1 + 1023) // 1024) * 1024
        cnt_v[...] = jnp.zeros((16,), jnp.int32) + cp0
        pltpu.sync_copy(cnt_v, counts.at[s * 2 + 0])
        cnt_v[...] = jnp.zeros((16,), jnp.int32) + cp1
        pltpu.sync_copy(cnt_v, counts.at[s * 2 + 1])
        pltpu.sync_copy(l0s, souts.at[s * 2 + 0])
        pltpu.sync_copy(l0d, douts.at[s * 2 + 0])
        pltpu.sync_copy(l1s, souts.at[s * 2 + 1])
        pltpu.sync_copy(l1d, douts.at[s * 2 + 1])


def _agg_body(u_hbm, src_hbm, dst_hbm, cnt_hbm, zeros_hbm, out_hbm, *refs):
    src_v, dst_v, cnt_v = refs[0], refs[1], refs[2]
    bufs = refs[3:3 + KB]
    gsem = refs[3 + KB:3 + 2 * KB]
    ssem = refs[3 + 2 * KB:3 + 3 * KB]
    agg_sh = refs[3 + 3 * KB]
    c = lax.axis_index("c")
    s = lax.axis_index("s")
    w = s * 2 + c
    pltpu.sync_copy(cnt_hbm.at[w], cnt_v)
    pltpu.sync_copy(src_hbm.at[w], src_v)
    pltpu.sync_copy(dst_hbm.at[w], dst_v)
    ngroups = jnp.max(cnt_v[...]) // (K * KB)

    pltpu.sync_copy(zeros_hbm, bufs[0])
    zcps = [
        pltpu.async_copy(
            bufs[0].at[pl.ds(0, WBC)],
            agg_sh.at[pl.ds(s * (3 * WBC) + r * WBC, WBC)], gsem[r % KB])
        for r in range(3)
    ]
    for cp in zcps:
        cp.wait()

    @pl.when(s == 0)
    def _():
        pltpu.async_copy(
            bufs[0].at[pl.ds(0, 8)],
            agg_sh.at[pl.ds(16 * 3 * WBC, 8)], ssem[0]).wait()
    plsc.subcore_barrier()

    def group(g, carry):
        base = g * KB
        gcps = [
            pltpu.async_copy(
                u_hbm.at[src_v.at[base + b]], bufs[b], gsem[b])
            for b in range(KB)
        ]
        scps = []
        for b in range(KB):
            gcps[b].wait()
            scps.append(pltpu.async_copy(
                bufs[b], agg_sh.at[dst_v.at[base + b]], ssem[b], add=True))
        for cp in scps:
            cp.wait()
        return carry
    lax.fori_loop(0, ngroups, group, 0)

    plsc.subcore_barrier()

    for r in range(3):
        b = r % KB
        pltpu.sync_copy(
            agg_sh.at[pl.ds(s * (3 * WBC) + r * WBC, WBC)],
            bufs[b].at[pl.ds(0, WBC)])
        pltpu.sync_copy(
            bufs[b].at[pl.ds(0, WBC)],
            out_hbm.at[pl.ds(c * TROWS + s * (3 * WBC) + r * WBC, WBC)])

    @pl.when(s == 0)
    def _():
        pltpu.sync_copy(
            agg_sh.at[pl.ds(16 * 3 * WBC, 8)], bufs[0].at[pl.ds(0, 8)])
        pltpu.sync_copy(
            bufs[0].at[pl.ds(0, 8)],
            out_hbm.at[pl.ds(c * TROWS + 16 * 3 * WBC, 8)])


@functools.cache
def _sc_kernels():
    mesh = plsc.VectorSubcoreMesh(
        core_axis_name="c", subcore_axis_name="s",
        num_cores=NC, num_subcores=NS)
    params = pltpu.CompilerParams(
        use_tc_tiling_on_sc=False, needs_layout_passes=False)
    deg_k = pl.kernel(
        _deg_body,
        out_type=jax.ShapeDtypeStruct((NC * NPAD, 16), jnp.float32),
        mesh=mesh,
        compiler_params=params,
        scratch_types=[
            pltpu.VMEM((NCH, K), jnp.int32),
            pltpu.VMEM((K, 16), jnp.float32),
            pltpu.VMEM((RPT, 16), jnp.float32),
            pltpu.VMEM_SHARED((NPAD, 16), jnp.float32),
        ],
    )
    route_k = pl.kernel(
        _route_body,
        out_type=[
            jax.ShapeDtypeStruct((NS * 2, CAP), jnp.int32),
            jax.ShapeDtypeStruct((NS * 2, CAP), jnp.int32),
            jax.ShapeDtypeStruct((NS * 2, 16), jnp.int32),
        ],
        mesh=mesh,
        compiler_params=params,
        scratch_types=[
            pltpu.VMEM((EPT,), jnp.int32),
            pltpu.VMEM((EPT,), jnp.int32),
            pltpu.VMEM((CAP,), jnp.int32),
            pltpu.VMEM((CAP,), jnp.int32),
            pltpu.VMEM((CAP,), jnp.int32),
            pltpu.VMEM((CAP,), jnp.int32),
            pltpu.VMEM((16,), jnp.int32),
        ],
    )
    agg_k = pl.kernel(
        _agg_body,
        out_type=jax.ShapeDtypeStruct((NC * TROWS, D), jnp.float32),
        mesh=mesh,
        compiler_params=params,
        scratch_types=(
            [pltpu.VMEM((NCAP, K), jnp.int32),
             pltpu.VMEM((NCAP, K), jnp.int32),
             pltpu.VMEM((16,), jnp.int32)]
            + [pltpu.VMEM((K, D), jnp.float32)] * KB
            + [pltpu.SemaphoreType.DMA] * (2 * KB)
            + [pltpu.VMEM_SHARED((TROWS, D), jnp.float32)]
        ),
    )
    return deg_k, route_k, agg_k


def _deg_kernel(dstp):
    ones = jnp.ones((K, 16), jnp.float32)
    zeros = jnp.zeros((RPT, 16), jnp.float32)
    out = _sc_kernels()[0](dstp, ones, zeros)
    return out.reshape(NC, NPAD, 16)


def _route_kernel(srcf, dstf):
    souts, douts, counts = _sc_kernels()[1](srcf, dstf)
    return (souts.reshape(NS * 2, NCAP, K),
            douts.reshape(NS * 2, NCAP, K), counts)


def _agg_kernel(u, souts, douts, counts):
    zeros = jnp.zeros((K, D), jnp.float32)
    out = _sc_kernels()[2](u, souts, douts, counts, zeros)
    return out.reshape(NC, TROWS, D)


def _m0_body(x_ref, w_ref, degp_ref, u_ref, dinv_ref):
    deg = degp_ref[0, :, 0] + degp_ref[1, :, 0] + 1.0
    dv = lax.rsqrt(deg)
    dinv_ref[...] = dv[:, None]
    u_ref[...] = jnp.dot(x_ref[...], w_ref[...],
                         preferred_element_type=jnp.float32) * dv[:, None]


def _m0_call(x, w, degp):
    return pl.pallas_call(
        _m0_body,
        grid=(NBLK,),
        in_specs=[
            pl.BlockSpec((NB, D), lambda i: (i, 0)),
            pl.BlockSpec((D, D), lambda i: (0, 0)),
            pl.BlockSpec((NC, NB, 16), lambda i: (0, i, 0)),
        ],
        out_specs=[
            pl.BlockSpec((NB, D), lambda i: (i, 0)),
            pl.BlockSpec((NB, 1), lambda i: (i, 0)),
        ],
        out_shape=[
            jax.ShapeDtypeStruct((N, D), jnp.float32),
            jax.ShapeDtypeStruct((N, 1), jnp.float32),
        ],
    )(x, w, degp)


def _comb_body(aggp_ref, u_ref, dinv_ref, b_ref, y_ref, st_ref):
    ci = pl.program_id(0)
    i = pl.program_id(1)
    y = dinv_ref[...] * (aggp_ref[0] + u_ref[...]) + b_ref[...]
    y_ref[...] = y
    ssum = jnp.sum(y, axis=0, keepdims=True)
    ssq = jnp.sum(y * y, axis=0, keepdims=True)
    st = jnp.concatenate([ssum, ssq], axis=0)
    first = jnp.logical_and(ci == 0, i == 0)

    @pl.when(first)
    def _():
        st_ref[...] = st

    @pl.when(jnp.logical_not(first))
    def _():
        st_ref[...] += st


def _comb_call(aggp, u, dinv, b):
    return pl.pallas_call(
        _comb_body,
        grid=(NC, NBLK2),
        in_specs=[
            pl.BlockSpec((1, NB2, D), lambda c, i: (c, i, 0)),
            pl.BlockSpec((NB2, D), lambda c, i: (c * NBLK2 + i, 0)),
            pl.BlockSpec((NB2, 1), lambda c, i: (c * NBLK2 + i, 0)),
            pl.BlockSpec((1, D), lambda c, i: (0, 0)),
        ],
        out_specs=[
            pl.BlockSpec((NB2, D), lambda c, i: (c * NBLK2 + i, 0)),
            pl.BlockSpec((2, D), lambda c, i: (0, 0)),
        ],
        out_shape=[
            jax.ShapeDtypeStruct((N, D), jnp.float32),
            jax.ShapeDtypeStruct((2, D), jnp.float32),
        ],
    )(aggp, u, dinv, b.reshape(1, D))


def _m_body(y_ref, st_ref, g_ref, be_ref, w_ref, dinv_ref, u_ref):
    mean = st_ref[0:1, :] * (1.0 / N)
    var = st_ref[1:2, :] * (1.0 / N) - mean * mean
    a = g_ref[...] * lax.rsqrt(var + EPS)
    cshift = be_ref[...] - mean * a
    t = jnp.maximum(y_ref[...] * a + cshift, 0.0)
    u_ref[...] = jnp.dot(t, w_ref[...],
                         preferred_element_type=jnp.float32) * dinv_ref[...]


def _m_call(y, st, g, be, w, dinv):
    return pl.pallas_call(
        _m_body,
        grid=(NBLK,),
        in_specs=[
            pl.BlockSpec((NB, D), lambda i: (i, 0)),
            pl.BlockSpec((2, D), lambda i: (0, 0)),
            pl.BlockSpec((1, D), lambda i: (0, 0)),
            pl.BlockSpec((1, D), lambda i: (0, 0)),
            pl.BlockSpec((D, D), lambda i: (0, 0)),
            pl.BlockSpec((NB, 1), lambda i: (i, 0)),
        ],
        out_specs=pl.BlockSpec((NB, D), lambda i: (i, 0)),
        out_shape=jax.ShapeDtypeStruct((N, D), jnp.float32),
    )(y, st, g.reshape(1, D), be.reshape(1, D), w, dinv)


def _p_body(y_ref, st_ref, g_ref, be_ref, batch_ref, lw_ref, lb_ref,
            out_ref, psum, cnt):
    i = pl.program_id(0)
    mean = st_ref[0:1, :] * (1.0 / N)
    var = st_ref[1:2, :] * (1.0 / N) - mean * mean
    a = g_ref[...] * lax.rsqrt(var + EPS)
    cshift = be_ref[...] - mean * a
    t = jnp.maximum(y_ref[...] * a + cshift, 0.0)
    bt = batch_ref[0]
    gi = lax.broadcasted_iota(jnp.int32, (G, NB), 0)
    sel = jnp.where(gi == bt, 1.0, 0.0)

    @pl.when(i == 0)
    def _():
        psum[...] = jnp.zeros_like(psum)
        cnt[...] = jnp.zeros_like(cnt)

    psum[...] += jnp.dot(sel, t, preferred_element_type=jnp.float32)
    cnt[...] += jnp.sum(sel, axis=1, keepdims=True)

    @pl.when(i == pl.num_programs(0) - 1)
    def _():
        pooled = psum[...] / jnp.maximum(cnt[...], 1.0)
        out_ref[...] = jnp.dot(pooled, lw_ref[...],
                               preferred_element_type=jnp.float32) + lb_ref[...]


def _p_call(y, st, g, be, batch3, lw, lb):
    return pl.pallas_call(
        _p_body,
        grid=(NBLK,),
        in_specs=[
            pl.BlockSpec((NB, D), lambda i: (i, 0)),
            pl.BlockSpec((2, D), lambda i: (0, 0)),
            pl.BlockSpec((1, D), lambda i: (0, 0)),
            pl.BlockSpec((1, D), lambda i: (0, 0)),
            pl.BlockSpec((1, 1, NB), lambda i: (i, 0, 0)),
            pl.BlockSpec((D, D), lambda i: (0, 0)),
            pl.BlockSpec((1, D), lambda i: (0, 0)),
        ],
        out_specs=pl.BlockSpec((G, D), lambda i: (0, 0)),
        out_shape=jax.ShapeDtypeStruct((G, D), jnp.float32),
        scratch_shapes=[
            pltpu.VMEM((G, D), jnp.float32),
            pltpu.VMEM((G, 1), jnp.float32),
        ],
    )(y, st, g.reshape(1, D), be.reshape(1, D), batch3, lw,
      lb.reshape(1, D))


def kernel(x, edge_index, batch, W0, b0, g0, be0, W1, b1, g1, be1,
           W2, b2, g2, be2, lin_W, lin_b):
    src = edge_index[0]
    dst = edge_index[1]
    pad = EPAD - E
    srcf = jnp.concatenate(
        [src, jnp.zeros((pad,), jnp.int32)]).reshape(NS, EPT)
    dstf = jnp.concatenate(
        [dst, jnp.full((pad,), N, jnp.int32)]).reshape(NS, EPT)
    batch3 = batch.reshape(NBLK, 1, NB)

    souts, douts, counts = _route_kernel(srcf, dstf)
    degp = _deg_kernel(dstf.reshape(NS, NCH, K))
    u, dinv = _m0_call(x, W0, degp)

    Wstack = jnp.stack([W0, W1, W2])
    bstack = jnp.stack([b0, b1, b2])
    gstack = jnp.stack([g0, g1, g2])
    bestack = jnp.stack([be0, be1, be2])

    def layer(i, carry):
        u, _, _ = carry
        aggp = _agg_kernel(u, souts, douts, counts)
        y, st = _comb_call(aggp, u, dinv, jnp.take(bstack, i, axis=0))
        wnext = jnp.take(Wstack, jnp.minimum(i + 1, 2), axis=0)
        u2 = _m_call(y, st, jnp.take(gstack, i, axis=0),
                     jnp.take(bestack, i, axis=0), wnext, dinv)
        return (u2, y, st)

    init = (u, jnp.zeros((N, D), jnp.float32), jnp.zeros((2, D), jnp.float32))
    _, y, st = lax.fori_loop(0, 3, layer, init)
    return _p_call(y, st, g2, be2, batch3, lin_W, lin_b)


# speedup vs baseline: 3.3916x; 3.3916x over previous
import functools

import jax
import jax.numpy as jnp
from jax import lax
from jax.experimental import pallas as pl
from jax.experimental.pallas import tpu as pltpu
from jax.experimental.pallas import tpu_sc as plsc

N = 10000
E = 320000
D = 128
HD = D // 2
G = 64
EPS = 1e-5

NC = 2
NS = 16
NPAD = 10240
K = 128
NCH = 160
EPT = NCH * K
EPAD = NS * EPT
RPT = NPAD // NS
DUMMY = N + 16
NB = 400
NBLK = N // NB


def _deg_body(dst_hbm, ones_hbm, zeros_hbm, out_hbm,
              dst_v, ones_v, buf_v, deg_sh):
    c = lax.axis_index("c")
    s = lax.axis_index("s")
    pltpu.sync_copy(dst_hbm.at[s], dst_v)
    pltpu.sync_copy(ones_hbm, ones_v)
    pltpu.sync_copy(zeros_hbm, buf_v)

    pltpu.sync_copy(buf_v, deg_sh.at[pl.ds(s * RPT, RPT)])
    plsc.subcore_barrier()

    half = NCH // NC

    def chunk(j, carry):
        pltpu.sync_copy(ones_v, deg_sh.at[dst_v.at[j]], add=True)
        return carry
    lax.fori_loop(c * half, (c + 1) * half, chunk, 0)
    plsc.subcore_barrier()

    pltpu.sync_copy(deg_sh.at[pl.ds(s * RPT, RPT)], buf_v)
    pltpu.sync_copy(buf_v, out_hbm.at[pl.ds(c * NPAD + s * RPT, RPT)])


KB = 4


def _agg_body(u_hbm, src_hbm, dst_hbm, zeros_hbm, out_hbm, *refs):
    src_v, dst_v = refs[0], refs[1]
    bufs = refs[2:2 + KB]
    gsem = refs[2 + KB:2 + 2 * KB]
    ssem = refs[2 + 2 * KB:2 + 3 * KB]
    agg_sh = refs[2 + 3 * KB]
    c = lax.axis_index("c")
    s = lax.axis_index("s")
    w = c * NS + s
    pltpu.sync_copy(src_hbm.at[w], src_v)
    pltpu.sync_copy(dst_hbm.at[s], dst_v)

    nwb = RPT // K
    pltpu.sync_copy(zeros_hbm, bufs[0])
    zcps = [
        pltpu.async_copy(
            bufs[0], agg_sh.at[pl.ds(s * RPT + r * K, K)], gsem[r % KB])
        for r in range(nwb)
    ]
    for cp in zcps:
        cp.wait()
    plsc.subcore_barrier()

    def group(g, carry):
        base = g * KB
        gcps = [
            pltpu.async_copy(
                u_hbm.at[src_v.at[base + b]], bufs[b], gsem[b])
            for b in range(KB)
        ]
        scps = []
        for b in range(KB):
            gcps[b].wait()
            scps.append(pltpu.async_copy(
                bufs[b], agg_sh.at[dst_v.at[base + b]], ssem[b], add=True))
        for cp in scps:
            cp.wait()
        return carry
    lax.fori_loop(0, NCH // KB, group, 0)

    plsc.subcore_barrier()

    for base in range(0, nwb, KB):
        cnt = min(KB, nwb - base)
        rcps = [
            pltpu.async_copy(
                agg_sh.at[pl.ds(s * RPT + (base + i) * K, K)],
                bufs[i], gsem[i])
            for i in range(cnt)
        ]
        wcps = []
        for i in range(cnt):
            rcps[i].wait()
            wcps.append(pltpu.async_copy(
                bufs[i],
                out_hbm.at[pl.ds(c * NPAD + s * RPT + (base + i) * K, K)],
                ssem[i]))
        for cp in wcps:
            cp.wait()


@functools.cache
def _sc_kernels():
    mesh = plsc.VectorSubcoreMesh(
        core_axis_name="c", subcore_axis_name="s",
        num_cores=NC, num_subcores=NS)
    deg_k = pl.kernel(
        _deg_body,
        out_type=jax.ShapeDtypeStruct((NC * NPAD, 16), jnp.float32),
        mesh=mesh,
        compiler_params=pltpu.CompilerParams(use_tc_tiling_on_sc=False),
        scratch_types=[
            pltpu.VMEM((NCH, K), jnp.int32),
            pltpu.VMEM((K, 16), jnp.float32),
            pltpu.VMEM((RPT, 16), jnp.float32),
            pltpu.VMEM_SHARED((NPAD, 16), jnp.float32),
        ],
    )
    agg_k = pl.kernel(
        _agg_body,
        out_type=jax.ShapeDtypeStruct((NC * NPAD, HD), jnp.float32),
        mesh=mesh,
        compiler_params=pltpu.CompilerParams(use_tc_tiling_on_sc=False),
        scratch_types=(
            [pltpu.VMEM((NCH, K), jnp.int32),
             pltpu.VMEM((NCH, K), jnp.int32)]
            + [pltpu.VMEM((K, HD), jnp.float32)] * KB
            + [pltpu.SemaphoreType.DMA] * (2 * KB)
            + [pltpu.VMEM_SHARED((NPAD, HD), jnp.float32)]
        ),
    )
    return deg_k, agg_k


def _deg_kernel(dstp):
    ones = jnp.ones((K, 16), jnp.float32)
    zeros = jnp.zeros((RPT, 16), jnp.float32)
    out = _sc_kernels()[0](dstp, ones, zeros)
    return out.reshape(NC, NPAD, 16)


def _agg_kernel(u, src2, dstp):
    zeros = jnp.zeros((K, HD), jnp.float32)
    out = _sc_kernels()[1](u.reshape(NC * N, HD), src2, dstp, zeros)
    return out.reshape(NC, NPAD, HD)


def _m0_body(x_ref, w_ref, degp_ref, u_ref, dinv_ref):
    deg = degp_ref[0, :, 0] + degp_ref[1, :, 0] + 1.0
    dv = lax.rsqrt(deg)
    dinv_ref[...] = dv[:, None]
    u = jnp.dot(x_ref[...], w_ref[...],
                preferred_element_type=jnp.float32) * dv[:, None]
    u_ref[0] = u[:, :HD]
    u_ref[1] = u[:, HD:]


def _m0_call(x, w, degp):
    return pl.pallas_call(
        _m0_body,
        grid=(NBLK,),
        in_specs=[
            pl.BlockSpec((NB, D), lambda i: (i, 0)),
            pl.BlockSpec((D, D), lambda i: (0, 0)),
            pl.BlockSpec((NC, NB, 16), lambda i: (0, i, 0)),
        ],
        out_specs=[
            pl.BlockSpec((NC, NB, HD), lambda i: (0, i, 0)),
            pl.BlockSpec((NB, 1), lambda i: (i, 0)),
        ],
        out_shape=[
            jax.ShapeDtypeStruct((NC, N, HD), jnp.float32),
            jax.ShapeDtypeStruct((N, 1), jnp.float32),
        ],
    )(x, w, degp)


def _comb_body(aggp_ref, u_ref, dinv_ref, b_ref, y_ref, st_ref):
    i = pl.program_id(0)
    agg = jnp.concatenate([aggp_ref[0], aggp_ref[1]], axis=1)
    u = jnp.concatenate([u_ref[0], u_ref[1]], axis=1)
    y = dinv_ref[...] * (agg + u) + b_ref[...]
    y_ref[...] = y
    ssum = jnp.sum(y, axis=0, keepdims=True)
    ssq = jnp.sum(y * y, axis=0, keepdims=True)
    st = jnp.concatenate([ssum, ssq], axis=0)

    @pl.when(i == 0)
    def _():
        st_ref[...] = st

    @pl.when(i > 0)
    def _():
        st_ref[...] += st


def _comb_call(aggp, u, dinv, b):
    return pl.pallas_call(
        _comb_body,
        grid=(NBLK,),
        in_specs=[
            pl.BlockSpec((NC, NB, HD), lambda i: (0, i, 0)),
            pl.BlockSpec((NC, NB, HD), lambda i: (0, i, 0)),
            pl.BlockSpec((NB, 1), lambda i: (i, 0)),
            pl.BlockSpec((1, D), lambda i: (0, 0)),
        ],
        out_specs=[
            pl.BlockSpec((NB, D), lambda i: (i, 0)),
            pl.BlockSpec((2, D), lambda i: (0, 0)),
        ],
        out_shape=[
            jax.ShapeDtypeStruct((N, D), jnp.float32),
            jax.ShapeDtypeStruct((2, D), jnp.float32),
        ],
    )(aggp, u, dinv, b.reshape(1, D))


def _m_body(y_ref, st_ref, g_ref, be_ref, w_ref, dinv_ref, u_ref):
    mean = st_ref[0:1, :] * (1.0 / N)
    var = st_ref[1:2, :] * (1.0 / N) - mean * mean
    a = g_ref[...] * lax.rsqrt(var + EPS)
    cshift = be_ref[...] - mean * a
    t = jnp.maximum(y_ref[...] * a + cshift, 0.0)
    u = jnp.dot(t, w_ref[...],
                preferred_element_type=jnp.float32) * dinv_ref[...]
    u_ref[0] = u[:, :HD]
    u_ref[1] = u[:, HD:]


def _m_call(y, st, g, be, w, dinv):
    return pl.pallas_call(
        _m_body,
        grid=(NBLK,),
        in_specs=[
            pl.BlockSpec((NB, D), lambda i: (i, 0)),
            pl.BlockSpec((2, D), lambda i: (0, 0)),
            pl.BlockSpec((1, D), lambda i: (0, 0)),
            pl.BlockSpec((1, D), lambda i: (0, 0)),
            pl.BlockSpec((D, D), lambda i: (0, 0)),
            pl.BlockSpec((NB, 1), lambda i: (i, 0)),
        ],
        out_specs=pl.BlockSpec((NC, NB, HD), lambda i: (0, i, 0)),
        out_shape=jax.ShapeDtypeStruct((NC, N, HD), jnp.float32),
    )(y, st, g.reshape(1, D), be.reshape(1, D), w, dinv)


def _p_body(y_ref, st_ref, g_ref, be_ref, batch_ref, lw_ref, lb_ref,
            out_ref, psum, cnt):
    i = pl.program_id(0)
    mean = st_ref[0:1, :] * (1.0 / N)
    var = st_ref[1:2, :] * (1.0 / N) - mean * mean
    a = g_ref[...] * lax.rsqrt(var + EPS)
    cshift = be_ref[...] - mean * a
    t = jnp.maximum(y_ref[...] * a + cshift, 0.0)
    bt = batch_ref[0]
    gi = lax.broadcasted_iota(jnp.int32, (G, NB), 0)
    sel = jnp.where(gi == bt, 1.0, 0.0)

    @pl.when(i == 0)
    def _():
        psum[...] = jnp.zeros_like(psum)
        cnt[...] = jnp.zeros_like(cnt)

    psum[...] += jnp.dot(sel, t, preferred_element_type=jnp.float32)
    cnt[...] += jnp.sum(sel, axis=1, keepdims=True)

    @pl.when(i == pl.num_programs(0) - 1)
    def _():
        pooled = psum[...] / jnp.maximum(cnt[...], 1.0)
        out_ref[...] = jnp.dot(pooled, lw_ref[...],
                               preferred_element_type=jnp.float32) + lb_ref[...]


def _p_call(y, st, g, be, batch3, lw, lb):
    return pl.pallas_call(
        _p_body,
        grid=(NBLK,),
        in_specs=[
            pl.BlockSpec((NB, D), lambda i: (i, 0)),
            pl.BlockSpec((2, D), lambda i: (0, 0)),
            pl.BlockSpec((1, D), lambda i: (0, 0)),
            pl.BlockSpec((1, D), lambda i: (0, 0)),
            pl.BlockSpec((1, 1, NB), lambda i: (i, 0, 0)),
            pl.BlockSpec((D, D), lambda i: (0, 0)),
            pl.BlockSpec((1, D), lambda i: (0, 0)),
        ],
        out_specs=pl.BlockSpec((G, D), lambda i: (0, 0)),
        out_shape=jax.ShapeDtypeStruct((G, D), jnp.float32),
        scratch_shapes=[
            pltpu.VMEM((G, D), jnp.float32),
            pltpu.VMEM((G, 1), jnp.float32),
        ],
    )(y, st, g.reshape(1, D), be.reshape(1, D), batch3, lw,
      lb.reshape(1, D))


def kernel(x, edge_index, batch, W0, b0, g0, be0, W1, b1, g1, be1,
           W2, b2, g2, be2, lin_W, lin_b):
    src = edge_index[0]
    dst = edge_index[1]
    pad = EPAD - E
    srcp = jnp.concatenate(
        [src, jnp.zeros((pad,), jnp.int32)]).reshape(1, NS, NCH, K)
    src2 = jnp.concatenate(
        [srcp, srcp + N], axis=0).reshape(NC * NS, NCH, K)
    dstp = jnp.concatenate(
        [dst, jnp.full((pad,), DUMMY, jnp.int32)]).reshape(NS, NCH, K)
    batch3 = batch.reshape(NBLK, 1, NB)

    degp = _deg_kernel(dstp)
    u, dinv = _m0_call(x, W0, degp)

    Ws = [W0, W1, W2]
    bs = [b0, b1, b2]
    gs = [g0, g1, g2]
    bes = [be0, be1, be2]
    y = None
    st = None
    for i in range(3):
        aggp = _agg_kernel(u, src2, dstp)
        y, st = _comb_call(aggp, u, dinv, bs[i])
        if i < 2:
            u = _m_call(y, st, gs[i], bes[i], Ws[i + 1], dinv)
    return _p_call(y, st, gs[2], bes[2], batch3, lin_W, lin_b)
